# Initial kernel scaffold; baseline (speedup 1.0000x reference)
import jax
import jax.numpy as jnp
from jax.experimental import pallas as pl


def kernel(x, edge_index, W1, b1, Wa1, a1, W2, b2, Wa2, a2, W3, b3):
    raise NotImplementedError("write your pallas kernel here")


# speedup vs baseline: 7.3748x; 7.3748x over previous
import jax
import jax.numpy as jnp
from jax import lax
from jax.experimental import pallas as pl
from jax.experimental.pallas import tpu as pltpu
from jax.experimental.pallas import tpu_sc as plsc

N = 10000
E = 320000
D = 128
NC = 2
NS = 16
NW = NC * NS
L = 16

GROUP = 128
GPW = 80
EPAD = NW * GPW * GROUP
NACC = 10240
RPT = NACC // NS
JUNK = N

_mesh = plsc.VectorSubcoreMesh(
    core_axis_name="c", subcore_axis_name="s", num_cores=NC, num_subcores=NS)


def _fill_rows(ref, nrows, ncols, value):
    vecs = ncols // L
    v = jnp.full((L,), value, jnp.float32)

    def body(i, _):
        r = i // vecs
        col = (i % vecs) * L
        ref[r, pl.ds(col, L)] = v
        return 0

    lax.fori_loop(0, nrows * vecs, body, 0)


def _prop_body(u, srcg, dstg, out, idx_s, idx_d, rows, acc, sem):
    c = lax.axis_index("c")
    s = lax.axis_index("s")
    w = c * NS + s
    _fill_rows(rows, GROUP, D, 0.0)
    base = s * RPT
    for k in range(RPT // GROUP):
        pltpu.sync_copy(rows, acc.at[pl.ds(base + k * GROUP, GROUP)])
    plsc.subcore_barrier()
    pltpu.sync_copy(srcg.at[pl.ds(w * GPW, GPW)], idx_s)
    pltpu.sync_copy(dstg.at[pl.ds(w * GPW, GPW)], idx_d)

    def body(g, _):
        pltpu.async_copy(u.at[idx_s.at[g]], rows, sem).wait()
        pltpu.sync_copy(rows, acc.at[idx_d.at[g]], add=True)
        return 0

    lax.fori_loop(0, GPW, body, 0)
    plsc.subcore_barrier()
    pltpu.sync_copy(acc.at[pl.ds(base, RPT)], out.at[c, pl.ds(base, RPT)])


_prop_kernel = pl.kernel(
    _prop_body,
    out_type=jax.ShapeDtypeStruct((NC, NACC, D), jnp.float32),
    mesh=_mesh,
    scratch_types=[
        pltpu.VMEM((GPW, GROUP), jnp.int32),
        pltpu.VMEM((GPW, GROUP), jnp.int32),
        pltpu.VMEM((GROUP, D), jnp.float32),
        pltpu.VMEM_SHARED((NACC, D), jnp.float32),
        pltpu.SemaphoreType.DMA,
    ],
)


RB = 2000
NBLK = N // RB


def _matT(a, w):
    return lax.dot_general(a, w, (((1,), (1,)), ((), ())),
                           preferred_element_type=jnp.float32)


def _dinv_of(degs):
    deg = degs[0, :, 0] + degs[1, :, 0] + 1.0
    return lax.rsqrt(deg)[:, None]


def _winv_of(odegs):
    od = odegs[0, :, 0] + odegs[1, :, 0]
    return jnp.where(od > 0.0, 1.0 / od, 0.0)[:, None]


def _tc_first_body(degs, x, w, uo):
    uo[...] = _matT(x[...], w[...]) * _dinv_of(degs[...])


def _tc_g2a_body(degs, odegs, parts, u, w, b, uo):
    y = _dinv_of(degs[...]) * (parts[0] + parts[1] + u[...]) + b[...]
    uo[...] = _matT(y, w[...]) * _winv_of(odegs[...])


def _tc_a2g_body(degs, parts, w, uo):
    y = parts[0] + parts[1]
    uo[...] = _matT(y, w[...]) * _dinv_of(degs[...])


def _tc_final_body(degs, parts, u, b, o):
    y = _dinv_of(degs[...]) * (parts[0] + parts[1] + u[...]) + b[...]
    m = jnp.max(y, axis=1, keepdims=True)
    lse = jnp.log(jnp.sum(jnp.exp(y - m), axis=1, keepdims=True)) + m
    o[...] = y - lse


_row_spec = pl.BlockSpec((RB, D), lambda i: (i, 0))
_mat_spec = pl.BlockSpec((D, D), lambda i: (0, 0))
_deg_spec = pl.BlockSpec((2, RB, L), lambda i: (0, i, 0))
_part_spec = pl.BlockSpec((2, RB, D), lambda i: (0, i, 0))
_bias_spec = pl.BlockSpec((1, D), lambda i: (0, 0))
_row_out = jax.ShapeDtypeStruct((N, D), jnp.float32)


def _tc_first(degs, x, w):
    return pl.pallas_call(
        _tc_first_body, grid=(NBLK,),
        in_specs=[_deg_spec, _row_spec, _mat_spec],
        out_specs=_row_spec, out_shape=_row_out)(degs, x, w)


def _tc_g2a(degs, odegs, parts, u, w, b):
    return pl.pallas_call(
        _tc_g2a_body, grid=(NBLK,),
        in_specs=[_deg_spec, _deg_spec, _part_spec, _row_spec, _mat_spec,
                  _bias_spec],
        out_specs=_row_spec, out_shape=_row_out)(degs, odegs, parts, u, w, b)


def _tc_a2g(degs, parts, w):
    return pl.pallas_call(
        _tc_a2g_body, grid=(NBLK,),
        in_specs=[_deg_spec, _part_spec, _mat_spec],
        out_specs=_row_spec, out_shape=_row_out)(degs, parts, w)


def _tc_final(degs, parts, u, b):
    return pl.pallas_call(
        _tc_final_body, grid=(NBLK,),
        in_specs=[_deg_spec, _part_spec, _row_spec, _bias_spec],
        out_specs=_row_spec, out_shape=_row_out)(degs, parts, u, b)


def kernel(x, edge_index, W1, b1, Wa1, a1, W2, b2, Wa2, a2, W3, b3):
    del a1, a2
    ei = edge_index.astype(jnp.int32)
    src = jnp.concatenate(
        [ei[0], jnp.zeros((EPAD - E,), jnp.int32)]).reshape(NW * GPW, GROUP)
    src_j = jnp.concatenate(
        [ei[0], jnp.full((EPAD - E,), JUNK, jnp.int32)]).reshape(NW * GPW, GROUP)
    dst = jnp.concatenate(
        [ei[1], jnp.zeros((EPAD - E,), jnp.int32)]).reshape(NW * GPW, GROUP)
    dst_j = jnp.concatenate(
        [ei[1], jnp.full((EPAD - E,), JUNK, jnp.int32)]).reshape(NW * GPW, GROUP)

    ones_nd = jnp.ones((N, D), jnp.float32)
    degs = _prop_kernel(ones_nd, src, dst_j)[:, :N, :L]
    odegs = _prop_kernel(ones_nd, dst, src_j)[:, :N, :L]

    b1 = b1.reshape(1, D)
    b2 = b2.reshape(1, D)
    b3 = b3.reshape(1, D)

    u1 = _tc_first(degs, x, W1)
    s1 = _prop_kernel(u1, src, dst_j)[:, :N, :]
    u2 = _tc_g2a(degs, odegs, s1, u1, Wa1, b1)
    s2 = _prop_kernel(u2, src, dst_j)[:, :N, :]
    u3 = _tc_a2g(degs, s2, W2)
    s3 = _prop_kernel(u3, src, dst_j)[:, :N, :]
    u4 = _tc_g2a(degs, odegs, s3, u3, Wa2, b2)
    s4 = _prop_kernel(u4, src, dst_j)[:, :N, :]
    u5 = _tc_a2g(degs, s4, W3)
    s5 = _prop_kernel(u5, src, dst_j)[:, :N, :]
    return _tc_final(degs, s5, u5, b3)


# speedup vs baseline: 8.5118x; 1.1542x over previous
import jax
import jax.numpy as jnp
from jax import lax
from jax.experimental import pallas as pl
from jax.experimental.pallas import tpu as pltpu
from jax.experimental.pallas import tpu_sc as plsc

N = 10000
E = 320000
D = 128
NC = 2
NS = 16
NW = NC * NS
L = 16

GROUP = 128
GPW = 80
EPAD = NW * GPW * GROUP
NACC = 10240
RPT = NACC // NS
JUNK = N

_mesh = plsc.VectorSubcoreMesh(
    core_axis_name="c", subcore_axis_name="s", num_cores=NC, num_subcores=NS)


def _fill_rows(ref, nrows, ncols, value):
    vecs = ncols // L
    v = jnp.full((L,), value, jnp.float32)

    def body(i, _):
        r = i // vecs
        col = (i % vecs) * L
        ref[r, pl.ds(col, L)] = v
        return 0

    lax.fori_loop(0, nrows * vecs, body, 0)


CH = 16


def _prop_body(u, srcg, dstg, out, is0, is1, id0, id1, r0, r1, acc, g0, g1):
    c = lax.axis_index("c")
    s = lax.axis_index("s")
    w = c * NS + s
    iss = (is0, is1)
    ids = (id0, id1)
    bufs = (r0, r1)
    gsems = (g0, g1)
    _fill_rows(r0, GROUP, D, 0.0)
    base = s * RPT
    for k in range(RPT // GROUP):
        pltpu.sync_copy(r0, acc.at[pl.ds(base + k * GROUP, GROUP)])
    plsc.subcore_barrier()

    def stage(ch):
        pltpu.sync_copy(srcg.at[pl.ds(w * GPW + ch * CH, CH)], iss[ch % 2])
        pltpu.sync_copy(dstg.at[pl.ds(w * GPW + ch * CH, CH)], ids[ch % 2])

    def start_gather(g):
        return pltpu.async_copy(
            u.at[iss[(g // CH) % 2].at[g % CH]], bufs[g % 2], gsems[g % 2])

    pending = {}
    stage(0)
    pending[0] = start_gather(0)
    pending[1] = start_gather(1)
    for g in range(GPW):
        b = g % 2
        pending[b].wait()
        pltpu.sync_copy(bufs[b], acc.at[ids[(g // CH) % 2].at[g % CH]],
                        add=True)
        nxt = g + 2
        if nxt < GPW:
            if nxt % CH == 0:
                stage(nxt // CH)
            pending[b] = start_gather(nxt)
    plsc.subcore_barrier()
    pltpu.sync_copy(acc.at[pl.ds(base, RPT)], out.at[c, pl.ds(base, RPT)])


_prop_kernel = pl.kernel(
    _prop_body,
    out_type=jax.ShapeDtypeStruct((NC, NACC, D), jnp.float32),
    mesh=_mesh,
    scratch_types=[
        pltpu.VMEM((CH, GROUP), jnp.int32),
        pltpu.VMEM((CH, GROUP), jnp.int32),
        pltpu.VMEM((CH, GROUP), jnp.int32),
        pltpu.VMEM((CH, GROUP), jnp.int32),
        pltpu.VMEM((GROUP, D), jnp.float32),
        pltpu.VMEM((GROUP, D), jnp.float32),
        pltpu.VMEM_SHARED((NACC, D), jnp.float32),
        pltpu.SemaphoreType.DMA,
        pltpu.SemaphoreType.DMA,
    ],
)


RB = 2000
NBLK = N // RB


def _matT(a, w):
    return lax.dot_general(a, w, (((1,), (1,)), ((), ())),
                           preferred_element_type=jnp.float32)


def _dinv_of(degs):
    deg = degs[0, :, 0] + degs[1, :, 0] + 1.0
    return lax.rsqrt(deg)[:, None]


def _winv_of(odegs):
    od = odegs[0, :, 0] + odegs[1, :, 0]
    return jnp.where(od > 0.0, 1.0 / od, 0.0)[:, None]


def _tc_first_body(degs, x, w, uo):
    uo[...] = _matT(x[...], w[...]) * _dinv_of(degs[...])


def _tc_g2a_body(degs, odegs, parts, u, w, b, uo):
    y = _dinv_of(degs[...]) * (parts[0] + parts[1] + u[...]) + b[...]
    uo[...] = _matT(y, w[...]) * _winv_of(odegs[...])


def _tc_a2g_body(degs, parts, w, uo):
    y = parts[0] + parts[1]
    uo[...] = _matT(y, w[...]) * _dinv_of(degs[...])


def _tc_final_body(degs, parts, u, b, o):
    y = _dinv_of(degs[...]) * (parts[0] + parts[1] + u[...]) + b[...]
    m = jnp.max(y, axis=1, keepdims=True)
    lse = jnp.log(jnp.sum(jnp.exp(y - m), axis=1, keepdims=True)) + m
    o[...] = y - lse


_row_spec = pl.BlockSpec((RB, D), lambda i: (i, 0))
_mat_spec = pl.BlockSpec((D, D), lambda i: (0, 0))
_deg_spec = pl.BlockSpec((2, RB, L), lambda i: (0, i, 0))
_part_spec = pl.BlockSpec((2, RB, D), lambda i: (0, i, 0))
_bias_spec = pl.BlockSpec((1, D), lambda i: (0, 0))
_row_out = jax.ShapeDtypeStruct((N, D), jnp.float32)


def _tc_first(degs, x, w):
    return pl.pallas_call(
        _tc_first_body, grid=(NBLK,),
        in_specs=[_deg_spec, _row_spec, _mat_spec],
        out_specs=_row_spec, out_shape=_row_out)(degs, x, w)


def _tc_g2a(degs, odegs, parts, u, w, b):
    return pl.pallas_call(
        _tc_g2a_body, grid=(NBLK,),
        in_specs=[_deg_spec, _deg_spec, _part_spec, _row_spec, _mat_spec,
                  _bias_spec],
        out_specs=_row_spec, out_shape=_row_out)(degs, odegs, parts, u, w, b)


def _tc_a2g(degs, parts, w):
    return pl.pallas_call(
        _tc_a2g_body, grid=(NBLK,),
        in_specs=[_deg_spec, _part_spec, _mat_spec],
        out_specs=_row_spec, out_shape=_row_out)(degs, parts, w)


def _tc_final(degs, parts, u, b):
    return pl.pallas_call(
        _tc_final_body, grid=(NBLK,),
        in_specs=[_deg_spec, _part_spec, _row_spec, _bias_spec],
        out_specs=_row_spec, out_shape=_row_out)(degs, parts, u, b)


def kernel(x, edge_index, W1, b1, Wa1, a1, W2, b2, Wa2, a2, W3, b3):
    del a1, a2
    ei = edge_index.astype(jnp.int32)
    src = jnp.concatenate(
        [ei[0], jnp.zeros((EPAD - E,), jnp.int32)]).reshape(NW * GPW, GROUP)
    src_j = jnp.concatenate(
        [ei[0], jnp.full((EPAD - E,), JUNK, jnp.int32)]).reshape(NW * GPW, GROUP)
    dst = jnp.concatenate(
        [ei[1], jnp.zeros((EPAD - E,), jnp.int32)]).reshape(NW * GPW, GROUP)
    dst_j = jnp.concatenate(
        [ei[1], jnp.full((EPAD - E,), JUNK, jnp.int32)]).reshape(NW * GPW, GROUP)

    ones_nd = jnp.ones((N, D), jnp.float32)
    degs = _prop_kernel(ones_nd, src, dst_j)[:, :N, :L]
    odegs = _prop_kernel(ones_nd, dst, src_j)[:, :N, :L]

    b1 = b1.reshape(1, D)
    b2 = b2.reshape(1, D)
    b3 = b3.reshape(1, D)

    u1 = _tc_first(degs, x, W1)
    s1 = _prop_kernel(u1, src, dst_j)[:, :N, :]
    u2 = _tc_g2a(degs, odegs, s1, u1, Wa1, b1)
    s2 = _prop_kernel(u2, src, dst_j)[:, :N, :]
    u3 = _tc_a2g(degs, s2, W2)
    s3 = _prop_kernel(u3, src, dst_j)[:, :N, :]
    u4 = _tc_g2a(degs, odegs, s3, u3, Wa2, b2)
    s4 = _prop_kernel(u4, src, dst_j)[:, :N, :]
    u5 = _tc_a2g(degs, s4, W3)
    s5 = _prop_kernel(u5, src, dst_j)[:, :N, :]
    return _tc_final(degs, s5, u5, b3)


# speedup vs baseline: 10.9293x; 1.2840x over previous
import jax
import jax.numpy as jnp
from jax import lax
from jax.experimental import pallas as pl
from jax.experimental.pallas import tpu as pltpu
from jax.experimental.pallas import tpu_sc as plsc

N = 10000
E = 320000
D = 128
NC = 2
NS = 16
NW = NC * NS
L = 16

GROUP = 128
GPW = 80
EPAD = NW * GPW * GROUP
NACC = 10240
RPT = NACC // NS
JUNK = N

_mesh = plsc.VectorSubcoreMesh(
    core_axis_name="c", subcore_axis_name="s", num_cores=NC, num_subcores=NS)


def _fill_rows(ref, nrows, ncols, value):
    vecs = ncols // L
    v = jnp.full((L,), value, jnp.float32)

    def body(i, _):
        r = i // vecs
        col = (i % vecs) * L
        ref[r, pl.ds(col, L)] = v
        return 0

    lax.fori_loop(0, nrows * vecs, body, 0)


CH = 16


def _prop_body(u, srcg, dstg, out, is0, is1, id0, id1, r0, r1, acc, g0, g1):
    c = lax.axis_index("c")
    s = lax.axis_index("s")
    w = c * NS + s
    iss = (is0, is1)
    ids = (id0, id1)
    bufs = (r0, r1)
    gsems = (g0, g1)
    _fill_rows(r0, GROUP, D, 0.0)
    base = s * RPT
    for k in range(RPT // GROUP):
        pltpu.sync_copy(r0, acc.at[pl.ds(base + k * GROUP, GROUP)])
    plsc.subcore_barrier()

    def stage(ch):
        pltpu.sync_copy(srcg.at[pl.ds(w * GPW + ch * CH, CH)], iss[ch % 2])
        pltpu.sync_copy(dstg.at[pl.ds(w * GPW + ch * CH, CH)], ids[ch % 2])

    def start_gather(g):
        return pltpu.async_copy(
            u.at[iss[(g // CH) % 2].at[g % CH]], bufs[g % 2], gsems[g % 2])

    pending = {}
    stage(0)
    pending[0] = start_gather(0)
    pending[1] = start_gather(1)
    for g in range(GPW):
        b = g % 2
        pending[b].wait()
        pltpu.sync_copy(bufs[b], acc.at[ids[(g // CH) % 2].at[g % CH]],
                        add=True)
        nxt = g + 2
        if nxt < GPW:
            if nxt % CH == 0:
                stage(nxt // CH)
            pending[b] = start_gather(nxt)
    plsc.subcore_barrier()
    pltpu.sync_copy(acc.at[pl.ds(base, RPT)], out.at[c, pl.ds(base, RPT)])


_prop_kernel = pl.kernel(
    _prop_body,
    out_type=jax.ShapeDtypeStruct((NC, NACC, D), jnp.float32),
    mesh=_mesh,
    scratch_types=[
        pltpu.VMEM((CH, GROUP), jnp.int32),
        pltpu.VMEM((CH, GROUP), jnp.int32),
        pltpu.VMEM((CH, GROUP), jnp.int32),
        pltpu.VMEM((CH, GROUP), jnp.int32),
        pltpu.VMEM((GROUP, D), jnp.float32),
        pltpu.VMEM((GROUP, D), jnp.float32),
        pltpu.VMEM_SHARED((NACC, D), jnp.float32),
        pltpu.SemaphoreType.DMA,
        pltpu.SemaphoreType.DMA,
    ],
)


def _ones_body(dstg, out, id0, id1, buf, acc, s0, s1):
    c = lax.axis_index("c")
    s = lax.axis_index("s")
    w = c * NS + s
    ids = (id0, id1)
    sems = (s0, s1)
    _fill_rows(buf, GROUP, D, 0.0)
    base = s * RPT
    for k in range(RPT // GROUP):
        pltpu.sync_copy(buf, acc.at[pl.ds(base + k * GROUP, GROUP)])
    _fill_rows(buf, GROUP, D, 1.0)
    plsc.subcore_barrier()

    pending = {}
    for g in range(GPW):
        b = g % 2
        if g % CH == 0:
            pltpu.sync_copy(dstg.at[pl.ds(w * GPW + g, CH)],
                            ids[(g // CH) % 2])
        if b in pending:
            pending[b].wait()
        pending[b] = pltpu.async_copy(
            buf, acc.at[ids[(g // CH) % 2].at[g % CH]], sems[b], add=True)
    pending[0].wait()
    pending[1].wait()
    plsc.subcore_barrier()
    pltpu.sync_copy(acc.at[pl.ds(base, RPT)], out.at[c, pl.ds(base, RPT)])


_ones_kernel = pl.kernel(
    _ones_body,
    out_type=jax.ShapeDtypeStruct((NC, NACC, D), jnp.float32),
    mesh=_mesh,
    scratch_types=[
        pltpu.VMEM((CH, GROUP), jnp.int32),
        pltpu.VMEM((CH, GROUP), jnp.int32),
        pltpu.VMEM((GROUP, D), jnp.float32),
        pltpu.VMEM_SHARED((NACC, D), jnp.float32),
        pltpu.SemaphoreType.DMA,
        pltpu.SemaphoreType.DMA,
    ],
)


RB = 2000
NBLK = N // RB


def _matT(a, w):
    return lax.dot_general(a, w, (((1,), (1,)), ((), ())),
                           preferred_element_type=jnp.float32)


def _dinv_of(degs):
    deg = degs[0, :, 0] + degs[1, :, 0] + 1.0
    return lax.rsqrt(deg)[:, None]


def _winv_of(odegs):
    od = odegs[0, :, 0] + odegs[1, :, 0]
    return jnp.where(od > 0.0, 1.0 / od, 0.0)[:, None]


def _tc_first_body(degs, x, w, uo):
    uo[...] = _matT(x[...], w[...]) * _dinv_of(degs[...])


def _tc_g2a_body(degs, odegs, parts, u, w, b, uo):
    y = _dinv_of(degs[...]) * (parts[0] + parts[1] + u[...]) + b[...]
    uo[...] = _matT(y, w[...]) * _winv_of(odegs[...])


def _tc_a2g_body(degs, parts, w, uo):
    y = parts[0] + parts[1]
    uo[...] = _matT(y, w[...]) * _dinv_of(degs[...])


def _tc_final_body(degs, parts, u, b, o):
    y = _dinv_of(degs[...]) * (parts[0] + parts[1] + u[...]) + b[...]
    m = jnp.max(y, axis=1, keepdims=True)
    lse = jnp.log(jnp.sum(jnp.exp(y - m), axis=1, keepdims=True)) + m
    o[...] = y - lse


_row_spec = pl.BlockSpec((RB, D), lambda i: (i, 0))
_mat_spec = pl.BlockSpec((D, D), lambda i: (0, 0))
_deg_spec = pl.BlockSpec((2, RB, L), lambda i: (0, i, 0))
_part_spec = pl.BlockSpec((2, RB, D), lambda i: (0, i, 0))
_bias_spec = pl.BlockSpec((1, D), lambda i: (0, 0))
_row_out = jax.ShapeDtypeStruct((N, D), jnp.float32)


def _tc_first(degs, x, w):
    return pl.pallas_call(
        _tc_first_body, grid=(NBLK,),
        in_specs=[_deg_spec, _row_spec, _mat_spec],
        out_specs=_row_spec, out_shape=_row_out)(degs, x, w)


def _tc_g2a(degs, odegs, parts, u, w, b):
    return pl.pallas_call(
        _tc_g2a_body, grid=(NBLK,),
        in_specs=[_deg_spec, _deg_spec, _part_spec, _row_spec, _mat_spec,
                  _bias_spec],
        out_specs=_row_spec, out_shape=_row_out)(degs, odegs, parts, u, w, b)


def _tc_a2g(degs, parts, w):
    return pl.pallas_call(
        _tc_a2g_body, grid=(NBLK,),
        in_specs=[_deg_spec, _part_spec, _mat_spec],
        out_specs=_row_spec, out_shape=_row_out)(degs, parts, w)


def _tc_final(degs, parts, u, b):
    return pl.pallas_call(
        _tc_final_body, grid=(NBLK,),
        in_specs=[_deg_spec, _part_spec, _row_spec, _bias_spec],
        out_specs=_row_spec, out_shape=_row_out)(degs, parts, u, b)


def kernel(x, edge_index, W1, b1, Wa1, a1, W2, b2, Wa2, a2, W3, b3):
    del a1, a2
    ei = edge_index.astype(jnp.int32)
    src = jnp.concatenate(
        [ei[0], jnp.zeros((EPAD - E,), jnp.int32)]).reshape(NW * GPW, GROUP)
    src_j = jnp.concatenate(
        [ei[0], jnp.full((EPAD - E,), JUNK, jnp.int32)]).reshape(NW * GPW, GROUP)
    dst_j = jnp.concatenate(
        [ei[1], jnp.full((EPAD - E,), JUNK, jnp.int32)]).reshape(NW * GPW, GROUP)

    degs = _ones_kernel(dst_j)[:, :N, :L]
    odegs = _ones_kernel(src_j)[:, :N, :L]

    b1 = b1.reshape(1, D)
    b2 = b2.reshape(1, D)
    b3 = b3.reshape(1, D)

    u1 = _tc_first(degs, x, W1)
    s1 = _prop_kernel(u1, src, dst_j)[:, :N, :]
    u2 = _tc_g2a(degs, odegs, s1, u1, Wa1, b1)
    s2 = _prop_kernel(u2, src, dst_j)[:, :N, :]
    u3 = _tc_a2g(degs, s2, W2)
    s3 = _prop_kernel(u3, src, dst_j)[:, :N, :]
    u4 = _tc_g2a(degs, odegs, s3, u3, Wa2, b2)
    s4 = _prop_kernel(u4, src, dst_j)[:, :N, :]
    u5 = _tc_a2g(degs, s4, W3)
    s5 = _prop_kernel(u5, src, dst_j)[:, :N, :]
    return _tc_final(degs, s5, u5, b3)


# speedup vs baseline: 31.7088x; 2.9013x over previous
import jax
import jax.numpy as jnp
from jax import lax
from jax.experimental import pallas as pl
from jax.experimental.pallas import tpu as pltpu
from jax.experimental.pallas import tpu_sc as plsc

N = 10000
E = 320000
D = 128
NC = 2
NS = 16
NW = NC * NS
L = 16

GROUP = 128
GPW = 80
EPAD = NW * GPW * GROUP
NACC = 10240
RPT = NACC // NS
JUNK = N

_mesh = plsc.VectorSubcoreMesh(
    core_axis_name="c", subcore_axis_name="s", num_cores=NC, num_subcores=NS)


def _fill_rows(ref, nrows, ncols, value):
    vecs = ncols // L
    v = jnp.full((L,), value, jnp.float32)

    def body(i, _):
        r = i // vecs
        col = (i % vecs) * L
        ref[r, pl.ds(col, L)] = v
        return 0

    lax.fori_loop(0, nrows * vecs, body, 0)


CH = 16


def _prop_body(u, srcg, dstg, out, is0, is1, id0, id1, r0, r1, acc, g0, g1):
    c = lax.axis_index("c")
    s = lax.axis_index("s")
    w = c * NS + s
    iss = (is0, is1)
    ids = (id0, id1)
    bufs = (r0, r1)
    gsems = (g0, g1)
    _fill_rows(r0, GROUP, D, 0.0)
    base = s * RPT
    for k in range(RPT // GROUP):
        pltpu.sync_copy(r0, acc.at[pl.ds(base + k * GROUP, GROUP)])
    plsc.subcore_barrier()

    def stage(ch):
        pltpu.sync_copy(srcg.at[pl.ds(w * GPW + ch * CH, CH)], iss[ch % 2])
        pltpu.sync_copy(dstg.at[pl.ds(w * GPW + ch * CH, CH)], ids[ch % 2])

    def start_gather(g):
        return pltpu.async_copy(
            u.at[iss[(g // CH) % 2].at[g % CH]], bufs[g % 2], gsems[g % 2])

    pending = {}
    stage(0)
    pending[0] = start_gather(0)
    pending[1] = start_gather(1)
    for g in range(GPW):
        b = g % 2
        pending[b].wait()
        pltpu.sync_copy(bufs[b], acc.at[ids[(g // CH) % 2].at[g % CH]],
                        add=True)
        nxt = g + 2
        if nxt < GPW:
            if nxt % CH == 0:
                stage(nxt // CH)
            pending[b] = start_gather(nxt)
    plsc.subcore_barrier()
    pltpu.sync_copy(acc.at[pl.ds(base, RPT)], out.at[c, pl.ds(base, RPT)])


_prop_kernel = pl.kernel(
    _prop_body,
    out_type=jax.ShapeDtypeStruct((NC, NACC, D), jnp.float32),
    mesh=_mesh,
    scratch_types=[
        pltpu.VMEM((CH, GROUP), jnp.int32),
        pltpu.VMEM((CH, GROUP), jnp.int32),
        pltpu.VMEM((CH, GROUP), jnp.int32),
        pltpu.VMEM((CH, GROUP), jnp.int32),
        pltpu.VMEM((GROUP, D), jnp.float32),
        pltpu.VMEM((GROUP, D), jnp.float32),
        pltpu.VMEM_SHARED((NACC, D), jnp.float32),
        pltpu.SemaphoreType.DMA,
        pltpu.SemaphoreType.DMA,
    ],
)


def _ones_body(dstg, out, id0, id1, buf, acc, s0, s1):
    c = lax.axis_index("c")
    s = lax.axis_index("s")
    w = c * NS + s
    ids = (id0, id1)
    sems = (s0, s1)
    _fill_rows(buf, GROUP, D, 0.0)
    base = s * RPT
    for k in range(RPT // GROUP):
        pltpu.sync_copy(buf, acc.at[pl.ds(base + k * GROUP, GROUP)])
    _fill_rows(buf, GROUP, D, 1.0)
    plsc.subcore_barrier()

    pending = {}
    for g in range(GPW):
        b = g % 2
        if g % CH == 0:
            pltpu.sync_copy(dstg.at[pl.ds(w * GPW + g, CH)],
                            ids[(g // CH) % 2])
        if b in pending:
            pending[b].wait()
        pending[b] = pltpu.async_copy(
            buf, acc.at[ids[(g // CH) % 2].at[g % CH]], sems[b], add=True)
    pending[0].wait()
    pending[1].wait()
    plsc.subcore_barrier()
    pltpu.sync_copy(acc.at[pl.ds(base, RPT)], out.at[c, pl.ds(base, RPT)])


_ones_kernel = pl.kernel(
    _ones_body,
    out_type=jax.ShapeDtypeStruct((NC, NACC, D), jnp.float32),
    mesh=_mesh,
    scratch_types=[
        pltpu.VMEM((CH, GROUP), jnp.int32),
        pltpu.VMEM((CH, GROUP), jnp.int32),
        pltpu.VMEM((GROUP, D), jnp.float32),
        pltpu.VMEM_SHARED((NACC, D), jnp.float32),
        pltpu.SemaphoreType.DMA,
        pltpu.SemaphoreType.DMA,
    ],
)


RB = 2000
NBLK = N // RB


def _matT(a, w):
    return lax.dot_general(a, w, (((1,), (1,)), ((), ())),
                           preferred_element_type=jnp.float32)


def _dinv_of(degs):
    deg = degs[0, :, 0] + degs[1, :, 0] + 1.0
    return lax.rsqrt(deg)[:, None]


def _winv_of(odegs):
    od = odegs[0, :, 0] + odegs[1, :, 0]
    return jnp.where(od > 0.0, 1.0 / od, 0.0)[:, None]


def _tc_first_body(degs, x, w, uo):
    uo[...] = _matT(x[...], w[...]) * _dinv_of(degs[...])


def _tc_g2a_body(degs, odegs, parts, u, w, b, uo):
    y = _dinv_of(degs[...]) * (parts[0] + parts[1] + u[...]) + b[...]
    uo[...] = _matT(y, w[...]) * _winv_of(odegs[...])


def _tc_a2g_body(degs, parts, w, uo):
    y = parts[0] + parts[1]
    uo[...] = _matT(y, w[...]) * _dinv_of(degs[...])


def _tc_final_body(degs, parts, u, b, o):
    y = _dinv_of(degs[...]) * (parts[0] + parts[1] + u[...]) + b[...]
    m = jnp.max(y, axis=1, keepdims=True)
    lse = jnp.log(jnp.sum(jnp.exp(y - m), axis=1, keepdims=True)) + m
    o[...] = y - lse


_row_spec = pl.BlockSpec((RB, D), lambda i: (i, 0))
_mat_spec = pl.BlockSpec((D, D), lambda i: (0, 0))
_deg_spec = pl.BlockSpec((2, RB, L), lambda i: (0, i, 0))
_part_spec = pl.BlockSpec((2, RB, D), lambda i: (0, i, 0))
_bias_spec = pl.BlockSpec((1, D), lambda i: (0, 0))
_row_out = jax.ShapeDtypeStruct((N, D), jnp.float32)


def _tc_first(degs, x, w):
    return pl.pallas_call(
        _tc_first_body, grid=(NBLK,),
        in_specs=[_deg_spec, _row_spec, _mat_spec],
        out_specs=_row_spec, out_shape=_row_out)(degs, x, w)


def _tc_g2a(degs, odegs, parts, u, w, b):
    return pl.pallas_call(
        _tc_g2a_body, grid=(NBLK,),
        in_specs=[_deg_spec, _deg_spec, _part_spec, _row_spec, _mat_spec,
                  _bias_spec],
        out_specs=_row_spec, out_shape=_row_out)(degs, odegs, parts, u, w, b)


def _tc_a2g(degs, parts, w):
    return pl.pallas_call(
        _tc_a2g_body, grid=(NBLK,),
        in_specs=[_deg_spec, _part_spec, _mat_spec],
        out_specs=_row_spec, out_shape=_row_out)(degs, parts, w)


def _tc_final(degs, parts, u, b):
    return pl.pallas_call(
        _tc_final_body, grid=(NBLK,),
        in_specs=[_deg_spec, _part_spec, _row_spec, _bias_spec],
        out_specs=_row_spec, out_shape=_row_out)(degs, parts, u, b)


def kernel(x, edge_index, W1, b1, Wa1, a1, W2, b2, Wa2, a2, W3, b3):
    del a1, a2
    ei = edge_index.astype(jnp.int32)
    pad = jnp.arange(EPAD - E, dtype=jnp.int32)
    src = jnp.concatenate(
        [ei[0], pad % N]).reshape(NW * GPW, GROUP)
    src_j = jnp.concatenate(
        [ei[0], JUNK + pad % (NACC - N)]).reshape(NW * GPW, GROUP)
    dst_j = jnp.concatenate(
        [ei[1], JUNK + pad % (NACC - N)]).reshape(NW * GPW, GROUP)

    degs = _ones_kernel(dst_j)[:, :N, :L]
    odegs = _ones_kernel(src_j)[:, :N, :L]

    b1 = b1.reshape(1, D)
    b2 = b2.reshape(1, D)
    b3 = b3.reshape(1, D)

    u1 = _tc_first(degs, x, W1)
    s1 = _prop_kernel(u1, src, dst_j)[:, :N, :]
    u2 = _tc_g2a(degs, odegs, s1, u1, Wa1, b1)
    s2 = _prop_kernel(u2, src, dst_j)[:, :N, :]
    u3 = _tc_a2g(degs, s2, W2)
    s3 = _prop_kernel(u3, src, dst_j)[:, :N, :]
    u4 = _tc_g2a(degs, odegs, s3, u3, Wa2, b2)
    s4 = _prop_kernel(u4, src, dst_j)[:, :N, :]
    u5 = _tc_a2g(degs, s4, W3)
    s5 = _prop_kernel(u5, src, dst_j)[:, :N, :]
    return _tc_final(degs, s5, u5, b3)
